# Initial kernel scaffold; baseline (speedup 1.0000x reference)
import functools

import jax
import jax.numpy as jnp
from jax import lax
from jax.experimental import pallas as pl
from jax.experimental.pallas import tpu as pltpu
from jax.experimental.pallas import tpu_sc as plsc

N, E, DIN, H, DOUT = 10000, 320000, 128, 64, 128

NC, NS = 2, 16
NW = NC * NS
EPT = E // NW
CHUNK = 80
NCHUNKS = EPT // CHUNK
ROWS_PT = N // NS
ZR = 125

_PREC = jax.lax.Precision.HIGHEST


def _dot(a, b):
    return jax.lax.dot_general(a, b, (((1,), (0,)), ((), ())),
                               precision=_PREC,
                               preferred_element_type=jnp.float32)


def _sc_segment_sum(p, src3, dst3, zeros_z):
    mesh = plsc.VectorSubcoreMesh(core_axis_name="c", subcore_axis_name="s")

    @functools.partial(
        pl.kernel,
        mesh=mesh,
        out_type=jax.ShapeDtypeStruct((NC, N, H), jnp.float32),
        scratch_types=[
            pltpu.VMEM((NCHUNKS, CHUNK), jnp.int32),
            pltpu.VMEM((NCHUNKS, CHUNK), jnp.int32),
            pltpu.VMEM((CHUNK, H), jnp.float32),
            pltpu.VMEM((ZR, H), jnp.float32),
            pltpu.VMEM_SHARED((N, H), jnp.float32),
            pltpu.SemaphoreType.DMA,
        ],
    )
    def k(p_hbm, src_hbm, dst_hbm, z_hbm, out_hbm, sbuf, dbuf, rows, zbuf,
          acc, sem):
        c = lax.axis_index("c")
        s = lax.axis_index("s")
        w = c * NS + s

        pltpu.sync_copy(z_hbm, zbuf)
        for j in range(ROWS_PT // ZR):
            pltpu.sync_copy(zbuf, acc.at[pl.ds(s * ROWS_PT + j * ZR, ZR)])

        pltpu.sync_copy(src_hbm.at[w], sbuf)
        pltpu.sync_copy(dst_hbm.at[w], dbuf)
        plsc.subcore_barrier()

        def body(j, _):
            pltpu.async_copy(p_hbm.at[sbuf.at[j]], rows, sem).wait()
            pltpu.sync_copy(rows, acc.at[dbuf.at[j]], add=True)
            return 0

        lax.fori_loop(0, NCHUNKS, body, 0)
        plsc.subcore_barrier()

        for j in range(ROWS_PT // ZR):
            r0 = s * ROWS_PT + j * ZR
            pltpu.sync_copy(acc.at[pl.ds(r0, ZR)], zbuf)
            pltpu.sync_copy(zbuf, out_hbm.at[c].at[pl.ds(r0, ZR)])

    return k(p, src3, dst3, zeros_z)


_BR = 2000


def _tc_project0(x, W1):
    def body(x_ref, w_ref, o_ref):
        o_ref[...] = _dot(x_ref[...], w_ref[...])

    return pl.pallas_call(
        body,
        grid=(N // _BR,),
        in_specs=[
            pl.BlockSpec((_BR, DIN), lambda i: (i, 0)),
            pl.BlockSpec((DIN, H), lambda i: (0, 0)),
        ],
        out_specs=pl.BlockSpec((_BR, H), lambda i: (i, 0)),
        out_shape=jax.ShapeDtypeStruct((N, H), jnp.float32),
    )(x, W1)


def _tc_block(p, acc2, b1, W2, b2, g, be, W1n):
    def body(p_ref, a_ref, b1_ref, w2_ref, b2_ref, g_ref, be_ref, w1n_ref,
             o_ref):
        t = p_ref[...] + a_ref[0] + a_ref[1] + b1_ref[...]
        t = jnp.maximum(t, 0.0)
        u = _dot(t, w2_ref[...]) + b2_ref[...]
        mu = jnp.mean(u, axis=-1, keepdims=True)
        var = jnp.mean((u - mu) ** 2, axis=-1, keepdims=True)
        v = (u - mu) / jnp.sqrt(var + 1e-5) * g_ref[...] + be_ref[...]
        v = jnp.maximum(v, 0.0)
        o_ref[...] = _dot(v, w1n_ref[...])

    return pl.pallas_call(
        body,
        grid=(N // _BR,),
        in_specs=[
            pl.BlockSpec((_BR, H), lambda i: (i, 0)),
            pl.BlockSpec((NC, _BR, H), lambda i: (0, i, 0)),
            pl.BlockSpec((1, H), lambda i: (0, 0)),
            pl.BlockSpec((H, H), lambda i: (0, 0)),
            pl.BlockSpec((1, H), lambda i: (0, 0)),
            pl.BlockSpec((1, H), lambda i: (0, 0)),
            pl.BlockSpec((1, H), lambda i: (0, 0)),
            pl.BlockSpec((H, H), lambda i: (0, 0)),
        ],
        out_specs=pl.BlockSpec((_BR, H), lambda i: (i, 0)),
        out_shape=jax.ShapeDtypeStruct((N, H), jnp.float32),
    )(p, acc2, b1.reshape(1, H), W2, b2.reshape(1, H), g.reshape(1, H),
      be.reshape(1, H), W1n)


def _tc_final(p, acc2, b1, W2, b2):
    def body(p_ref, a_ref, b1_ref, w2_ref, b2_ref, o_ref):
        t = p_ref[...] + a_ref[0] + a_ref[1] + b1_ref[...]
        t = jnp.maximum(t, 0.0)
        o_ref[...] = _dot(t, w2_ref[...]) + b2_ref[...]

    return pl.pallas_call(
        body,
        grid=(N // _BR,),
        in_specs=[
            pl.BlockSpec((_BR, H), lambda i: (i, 0)),
            pl.BlockSpec((NC, _BR, H), lambda i: (0, i, 0)),
            pl.BlockSpec((1, H), lambda i: (0, 0)),
            pl.BlockSpec((H, DOUT), lambda i: (0, 0)),
            pl.BlockSpec((1, DOUT), lambda i: (0, 0)),
        ],
        out_specs=pl.BlockSpec((_BR, DOUT), lambda i: (i, 0)),
        out_shape=jax.ShapeDtypeStruct((N, DOUT), jnp.float32),
    )(p, acc2, b1.reshape(1, H), W2, b2.reshape(1, DOUT))


def kernel(x, edge_index,
           W1_0, b1_0, W2_0, b2_0, g_0, be_0,
           W1_1, b1_1, W2_1, b2_1, g_1, be_1,
           W1_2, b1_2, W2_2, b2_2, g_2, be_2,
           W1_3, b1_3, W2_3, b2_3):
    src3 = edge_index[0].astype(jnp.int32).reshape(NW, NCHUNKS, CHUNK)
    dst3 = edge_index[1].astype(jnp.int32).reshape(NW, NCHUNKS, CHUNK)
    zeros_z = jnp.zeros((ZR, H), jnp.float32)

    p = _tc_project0(x, W1_0)
    acc2 = _sc_segment_sum(p, src3, dst3, zeros_z)
    p = _tc_block(p, acc2, b1_0, W2_0, b2_0, g_0, be_0, W1_1)
    acc2 = _sc_segment_sum(p, src3, dst3, zeros_z)
    p = _tc_block(p, acc2, b1_1, W2_1, b2_1, g_1, be_1, W1_2)
    acc2 = _sc_segment_sum(p, src3, dst3, zeros_z)
    p = _tc_block(p, acc2, b1_2, W2_2, b2_2, g_2, be_2, W1_3)
    acc2 = _sc_segment_sum(p, src3, dst3, zeros_z)
    return _tc_final(p, acc2, b1_3, W2_3, b2_3)


# speedup vs baseline: 8.1945x; 8.1945x over previous
import functools

import jax
import jax.numpy as jnp
from jax import lax
from jax.experimental import pallas as pl
from jax.experimental.pallas import tpu as pltpu
from jax.experimental.pallas import tpu_sc as plsc

N, E, DIN, H, DOUT = 10000, 320000, 128, 64, 128

NC, NS = 2, 16
NW = NC * NS
EPT = E // NW
CHUNK = 80
NCHUNKS = EPT // CHUNK
NPAD = 10240
ROWS_PT = NPAD // NS
ZR = 128

_PREC = jax.lax.Precision.HIGHEST


def _dot(a, b):
    return jax.lax.dot_general(a, b, (((1,), (0,)), ((), ())),
                               precision=_PREC,
                               preferred_element_type=jnp.float32)


def _sc_segment_sum(p, src3, dst3, zeros_z):
    mesh = plsc.VectorSubcoreMesh(core_axis_name="c", subcore_axis_name="s")

    @functools.partial(
        pl.kernel,
        mesh=mesh,
        compiler_params=pltpu.CompilerParams(use_tc_tiling_on_sc=False),
        out_type=jax.ShapeDtypeStruct((NC, NPAD, H), jnp.float32),
        scratch_types=[
            pltpu.VMEM((NCHUNKS, CHUNK), jnp.int32),
            pltpu.VMEM((NCHUNKS, CHUNK), jnp.int32),
            pltpu.VMEM((CHUNK, H), jnp.float32),
            pltpu.VMEM((ZR, H), jnp.float32),
            pltpu.VMEM_SHARED((NPAD, H), jnp.float32),
            pltpu.SemaphoreType.DMA,
        ],
    )
    def k(p_hbm, src_hbm, dst_hbm, z_hbm, out_hbm, sbuf, dbuf, rows, zbuf,
          acc, sem):
        c = lax.axis_index("c")
        s = lax.axis_index("s")
        w = c * NS + s

        pltpu.sync_copy(z_hbm, zbuf)
        for j in range(ROWS_PT // ZR):
            pltpu.sync_copy(zbuf, acc.at[pl.ds(s * ROWS_PT + j * ZR, ZR)])

        pltpu.sync_copy(src_hbm.at[w], sbuf)
        pltpu.sync_copy(dst_hbm.at[w], dbuf)
        plsc.subcore_barrier()

        def body(j, _):
            pltpu.async_copy(p_hbm.at[sbuf.at[j]], rows, sem).wait()
            pltpu.sync_copy(rows, acc.at[dbuf.at[j]], add=True)
            return 0

        lax.fori_loop(0, NCHUNKS, body, 0)
        plsc.subcore_barrier()

        for j in range(ROWS_PT // ZR):
            r0 = s * ROWS_PT + j * ZR
            pltpu.sync_copy(acc.at[pl.ds(r0, ZR)], zbuf)
            pltpu.sync_copy(zbuf, out_hbm.at[c].at[pl.ds(r0, ZR)])

    return k(p, src3, dst3, zeros_z)


_BR = 2000


def _tc_project0(x, W1):
    def body(x_ref, w_ref, o_ref):
        o_ref[...] = _dot(x_ref[...], w_ref[...])

    return pl.pallas_call(
        body,
        grid=(N // _BR,),
        in_specs=[
            pl.BlockSpec((_BR, DIN), lambda i: (i, 0)),
            pl.BlockSpec((DIN, H), lambda i: (0, 0)),
        ],
        out_specs=pl.BlockSpec((_BR, H), lambda i: (i, 0)),
        out_shape=jax.ShapeDtypeStruct((N, H), jnp.float32),
    )(x, W1)


def _tc_block(p, acc2, b1, W2, b2, g, be, W1n):
    def body(p_ref, a_ref, b1_ref, w2_ref, b2_ref, g_ref, be_ref, w1n_ref,
             o_ref):
        t = p_ref[...] + a_ref[0] + a_ref[1] + b1_ref[...]
        t = jnp.maximum(t, 0.0)
        u = _dot(t, w2_ref[...]) + b2_ref[...]
        mu = jnp.mean(u, axis=-1, keepdims=True)
        var = jnp.mean((u - mu) ** 2, axis=-1, keepdims=True)
        v = (u - mu) / jnp.sqrt(var + 1e-5) * g_ref[...] + be_ref[...]
        v = jnp.maximum(v, 0.0)
        o_ref[...] = _dot(v, w1n_ref[...])

    return pl.pallas_call(
        body,
        grid=(N // _BR,),
        in_specs=[
            pl.BlockSpec((_BR, H), lambda i: (i, 0)),
            pl.BlockSpec((NC, _BR, H), lambda i: (0, i, 0)),
            pl.BlockSpec((1, H), lambda i: (0, 0)),
            pl.BlockSpec((H, H), lambda i: (0, 0)),
            pl.BlockSpec((1, H), lambda i: (0, 0)),
            pl.BlockSpec((1, H), lambda i: (0, 0)),
            pl.BlockSpec((1, H), lambda i: (0, 0)),
            pl.BlockSpec((H, H), lambda i: (0, 0)),
        ],
        out_specs=pl.BlockSpec((_BR, H), lambda i: (i, 0)),
        out_shape=jax.ShapeDtypeStruct((N, H), jnp.float32),
    )(p, acc2, b1.reshape(1, H), W2, b2.reshape(1, H), g.reshape(1, H),
      be.reshape(1, H), W1n)


def _tc_final(p, acc2, b1, W2, b2):
    def body(p_ref, a_ref, b1_ref, w2_ref, b2_ref, o_ref):
        t = p_ref[...] + a_ref[0] + a_ref[1] + b1_ref[...]
        t = jnp.maximum(t, 0.0)
        o_ref[...] = _dot(t, w2_ref[...]) + b2_ref[...]

    return pl.pallas_call(
        body,
        grid=(N // _BR,),
        in_specs=[
            pl.BlockSpec((_BR, H), lambda i: (i, 0)),
            pl.BlockSpec((NC, _BR, H), lambda i: (0, i, 0)),
            pl.BlockSpec((1, H), lambda i: (0, 0)),
            pl.BlockSpec((H, DOUT), lambda i: (0, 0)),
            pl.BlockSpec((1, DOUT), lambda i: (0, 0)),
        ],
        out_specs=pl.BlockSpec((_BR, DOUT), lambda i: (i, 0)),
        out_shape=jax.ShapeDtypeStruct((N, DOUT), jnp.float32),
    )(p, acc2, b1.reshape(1, H), W2, b2.reshape(1, DOUT))


def kernel(x, edge_index,
           W1_0, b1_0, W2_0, b2_0, g_0, be_0,
           W1_1, b1_1, W2_1, b2_1, g_1, be_1,
           W1_2, b1_2, W2_2, b2_2, g_2, be_2,
           W1_3, b1_3, W2_3, b2_3):
    src3 = edge_index[0].astype(jnp.int32).reshape(NW, NCHUNKS, CHUNK)
    dst3 = edge_index[1].astype(jnp.int32).reshape(NW, NCHUNKS, CHUNK)
    zeros_z = jnp.zeros((ZR, H), jnp.float32)

    p = _tc_project0(x, W1_0)
    acc2 = _sc_segment_sum(p, src3, dst3, zeros_z)
    p = _tc_block(p, acc2, b1_0, W2_0, b2_0, g_0, be_0, W1_1)
    acc2 = _sc_segment_sum(p, src3, dst3, zeros_z)
    p = _tc_block(p, acc2, b1_1, W2_1, b2_1, g_1, be_1, W1_2)
    acc2 = _sc_segment_sum(p, src3, dst3, zeros_z)
    p = _tc_block(p, acc2, b1_2, W2_2, b2_2, g_2, be_2, W1_3)
    acc2 = _sc_segment_sum(p, src3, dst3, zeros_z)
    return _tc_final(p, acc2, b1_3, W2_3, b2_3)


# speedup vs baseline: 15.9239x; 1.9432x over previous
import functools

import jax
import jax.numpy as jnp
from jax import lax
from jax.experimental import pallas as pl
from jax.experimental.pallas import tpu as pltpu
from jax.experimental.pallas import tpu_sc as plsc

N, E, DIN, H, DOUT = 10000, 320000, 128, 64, 128

NC, NS = 2, 16
NW = NC * NS
EPT = E // NW
CHUNK = 100
NCHUNKS = EPT // CHUNK
NBUF = 4
NITER = NCHUNKS // NBUF
NPAD = 10240
ROWS_PT = NPAD // NS
ZR = 128

_PREC = jax.lax.Precision.HIGHEST


def _dot(a, b):
    return jax.lax.dot_general(a, b, (((1,), (0,)), ((), ())),
                               precision=_PREC,
                               preferred_element_type=jnp.float32)


def _sc_segment_sum(p, src3, dst3, zeros_z):
    mesh = plsc.VectorSubcoreMesh(core_axis_name="c", subcore_axis_name="s")

    @functools.partial(
        pl.kernel,
        mesh=mesh,
        compiler_params=pltpu.CompilerParams(use_tc_tiling_on_sc=False),
        out_type=jax.ShapeDtypeStruct((NC, NPAD, H), jnp.float32),
        scratch_types=[
            pltpu.VMEM((NCHUNKS, CHUNK), jnp.int32),
            pltpu.VMEM((NCHUNKS, CHUNK), jnp.int32),
            pltpu.VMEM((CHUNK, H), jnp.float32),
            pltpu.VMEM((CHUNK, H), jnp.float32),
            pltpu.VMEM((CHUNK, H), jnp.float32),
            pltpu.VMEM((CHUNK, H), jnp.float32),
            pltpu.VMEM((ZR, H), jnp.float32),
            pltpu.VMEM_SHARED((NPAD, H), jnp.float32),
            pltpu.SemaphoreType.DMA,
            pltpu.SemaphoreType.DMA,
            pltpu.SemaphoreType.DMA,
            pltpu.SemaphoreType.DMA,
        ],
    )
    def k(p_hbm, src_hbm, dst_hbm, z_hbm, out_hbm, sbuf, dbuf,
          r0, r1, r2, r3, zbuf, acc, g0, g1, g2, g3):
        rows = [r0, r1, r2, r3]
        gsem = [g0, g1, g2, g3]
        c = lax.axis_index("c")
        s = lax.axis_index("s")
        w = c * NS + s

        pltpu.sync_copy(z_hbm, zbuf)
        for j in range(ROWS_PT // ZR):
            pltpu.sync_copy(zbuf, acc.at[pl.ds(s * ROWS_PT + j * ZR, ZR)])

        pltpu.sync_copy(src_hbm.at[w], sbuf)
        pltpu.sync_copy(dst_hbm.at[w], dbuf)
        plsc.subcore_barrier()

        for b in range(NBUF):
            pltpu.async_copy(p_hbm.at[sbuf.at[b]], rows[b], gsem[b])

        def body(i, _):
            for b in range(NBUF):
                j = i * NBUF + b
                pltpu.make_async_copy(
                    p_hbm.at[sbuf.at[j]], rows[b], gsem[b]).wait()
                pltpu.sync_copy(rows[b], acc.at[dbuf.at[j]], add=True)

                @pl.when(i < NITER - 1)
                def _():
                    pltpu.async_copy(
                        p_hbm.at[sbuf.at[j + NBUF]], rows[b], gsem[b])
            return 0

        lax.fori_loop(0, NITER, body, 0)
        plsc.subcore_barrier()

        for j in range(ROWS_PT // ZR):
            r0 = s * ROWS_PT + j * ZR
            pltpu.sync_copy(acc.at[pl.ds(r0, ZR)], zbuf)
            pltpu.sync_copy(zbuf, out_hbm.at[c].at[pl.ds(r0, ZR)])

    return k(p, src3, dst3, zeros_z)


_BR = 2000


def _tc_project0(x, W1):
    def body(x_ref, w_ref, o_ref):
        o_ref[...] = _dot(x_ref[...], w_ref[...])

    return pl.pallas_call(
        body,
        grid=(N // _BR,),
        in_specs=[
            pl.BlockSpec((_BR, DIN), lambda i: (i, 0)),
            pl.BlockSpec((DIN, H), lambda i: (0, 0)),
        ],
        out_specs=pl.BlockSpec((_BR, H), lambda i: (i, 0)),
        out_shape=jax.ShapeDtypeStruct((N, H), jnp.float32),
    )(x, W1)


def _tc_block(p, acc2, b1, W2, b2, g, be, W1n):
    def body(p_ref, a_ref, b1_ref, w2_ref, b2_ref, g_ref, be_ref, w1n_ref,
             o_ref):
        t = p_ref[...] + a_ref[0] + a_ref[1] + b1_ref[...]
        t = jnp.maximum(t, 0.0)
        u = _dot(t, w2_ref[...]) + b2_ref[...]
        mu = jnp.mean(u, axis=-1, keepdims=True)
        var = jnp.mean((u - mu) ** 2, axis=-1, keepdims=True)
        v = (u - mu) / jnp.sqrt(var + 1e-5) * g_ref[...] + be_ref[...]
        v = jnp.maximum(v, 0.0)
        o_ref[...] = _dot(v, w1n_ref[...])

    return pl.pallas_call(
        body,
        grid=(N // _BR,),
        in_specs=[
            pl.BlockSpec((_BR, H), lambda i: (i, 0)),
            pl.BlockSpec((NC, _BR, H), lambda i: (0, i, 0)),
            pl.BlockSpec((1, H), lambda i: (0, 0)),
            pl.BlockSpec((H, H), lambda i: (0, 0)),
            pl.BlockSpec((1, H), lambda i: (0, 0)),
            pl.BlockSpec((1, H), lambda i: (0, 0)),
            pl.BlockSpec((1, H), lambda i: (0, 0)),
            pl.BlockSpec((H, H), lambda i: (0, 0)),
        ],
        out_specs=pl.BlockSpec((_BR, H), lambda i: (i, 0)),
        out_shape=jax.ShapeDtypeStruct((N, H), jnp.float32),
    )(p, acc2, b1.reshape(1, H), W2, b2.reshape(1, H), g.reshape(1, H),
      be.reshape(1, H), W1n)


def _tc_final(p, acc2, b1, W2, b2):
    def body(p_ref, a_ref, b1_ref, w2_ref, b2_ref, o_ref):
        t = p_ref[...] + a_ref[0] + a_ref[1] + b1_ref[...]
        t = jnp.maximum(t, 0.0)
        o_ref[...] = _dot(t, w2_ref[...]) + b2_ref[...]

    return pl.pallas_call(
        body,
        grid=(N // _BR,),
        in_specs=[
            pl.BlockSpec((_BR, H), lambda i: (i, 0)),
            pl.BlockSpec((NC, _BR, H), lambda i: (0, i, 0)),
            pl.BlockSpec((1, H), lambda i: (0, 0)),
            pl.BlockSpec((H, DOUT), lambda i: (0, 0)),
            pl.BlockSpec((1, DOUT), lambda i: (0, 0)),
        ],
        out_specs=pl.BlockSpec((_BR, DOUT), lambda i: (i, 0)),
        out_shape=jax.ShapeDtypeStruct((N, DOUT), jnp.float32),
    )(p, acc2, b1.reshape(1, H), W2, b2.reshape(1, DOUT))


def kernel(x, edge_index,
           W1_0, b1_0, W2_0, b2_0, g_0, be_0,
           W1_1, b1_1, W2_1, b2_1, g_1, be_1,
           W1_2, b1_2, W2_2, b2_2, g_2, be_2,
           W1_3, b1_3, W2_3, b2_3):
    src3 = edge_index[0].astype(jnp.int32).reshape(NW, NCHUNKS, CHUNK)
    dst3 = edge_index[1].astype(jnp.int32).reshape(NW, NCHUNKS, CHUNK)
    zeros_z = jnp.zeros((ZR, H), jnp.float32)

    p = _tc_project0(x, W1_0)
    acc2 = _sc_segment_sum(p, src3, dst3, zeros_z)
    p = _tc_block(p, acc2, b1_0, W2_0, b2_0, g_0, be_0, W1_1)
    acc2 = _sc_segment_sum(p, src3, dst3, zeros_z)
    p = _tc_block(p, acc2, b1_1, W2_1, b2_1, g_1, be_1, W1_2)
    acc2 = _sc_segment_sum(p, src3, dst3, zeros_z)
    p = _tc_block(p, acc2, b1_2, W2_2, b2_2, g_2, be_2, W1_3)
    acc2 = _sc_segment_sum(p, src3, dst3, zeros_z)
    return _tc_final(p, acc2, b1_3, W2_3, b2_3)


# speedup vs baseline: 16.0573x; 1.0084x over previous
import functools

import jax
import jax.numpy as jnp
from jax import lax
from jax.experimental import pallas as pl
from jax.experimental.pallas import tpu as pltpu
from jax.experimental.pallas import tpu_sc as plsc

N, E, DIN, H, DOUT = 10000, 320000, 128, 64, 128

NC, NS = 2, 16
NW = NC * NS
EPT = E // NW
CHUNK = 100
NCHUNKS = EPT // CHUNK
NBUF = 4
NITER = NCHUNKS // NBUF
NPAD = 10240
ROWS_PT = NPAD // NS
ZR = 128

_PREC = jax.lax.Precision.HIGHEST


def _dot(a, b):
    return jax.lax.dot_general(a, b, (((1,), (0,)), ((), ())),
                               precision=_PREC,
                               preferred_element_type=jnp.float32)


def _sc_segment_sum(p, src3, dst3, zeros_z):
    mesh = plsc.VectorSubcoreMesh(core_axis_name="c", subcore_axis_name="s")

    @functools.partial(
        pl.kernel,
        mesh=mesh,
        compiler_params=pltpu.CompilerParams(use_tc_tiling_on_sc=False),
        out_type=jax.ShapeDtypeStruct((NC, NPAD, H), jnp.float32),
        scratch_types=[
            pltpu.VMEM((NCHUNKS, CHUNK), jnp.int32),
            pltpu.VMEM((NCHUNKS, CHUNK), jnp.int32),
            pltpu.VMEM((CHUNK, H), jnp.float32),
            pltpu.VMEM((CHUNK, H), jnp.float32),
            pltpu.VMEM((CHUNK, H), jnp.float32),
            pltpu.VMEM((CHUNK, H), jnp.float32),
            pltpu.VMEM((ZR, H), jnp.float32),
            pltpu.VMEM_SHARED((NPAD, H), jnp.float32),
            pltpu.SemaphoreType.DMA,
            pltpu.SemaphoreType.DMA,
            pltpu.SemaphoreType.DMA,
            pltpu.SemaphoreType.DMA,
        ],
    )
    def k(p_hbm, src_hbm, dst_hbm, z_hbm, out_hbm, sbuf, dbuf,
          r0, r1, r2, r3, zbuf, acc, g0, g1, g2, g3):
        rows = [r0, r1, r2, r3]
        gsem = [g0, g1, g2, g3]
        c = lax.axis_index("c")
        s = lax.axis_index("s")
        w = c * NS + s

        pltpu.sync_copy(src_hbm.at[w], sbuf)
        pltpu.sync_copy(dst_hbm.at[w], dbuf)
        for b in range(NBUF):
            pltpu.async_copy(p_hbm.at[sbuf.at[b]], rows[b], gsem[b])

        pltpu.sync_copy(z_hbm, zbuf)
        for j in range(ROWS_PT // ZR):
            pltpu.sync_copy(zbuf, acc.at[pl.ds(s * ROWS_PT + j * ZR, ZR)])
        plsc.subcore_barrier()

        def body(i, _):
            for b in range(NBUF):
                j = i * NBUF + b
                pltpu.make_async_copy(
                    p_hbm.at[sbuf.at[j]], rows[b], gsem[b]).wait()
                pltpu.sync_copy(rows[b], acc.at[dbuf.at[j]], add=True)

                @pl.when(i < NITER - 1)
                def _():
                    pltpu.async_copy(
                        p_hbm.at[sbuf.at[j + NBUF]], rows[b], gsem[b])
            return 0

        lax.fori_loop(0, NITER, body, 0)
        plsc.subcore_barrier()

        for j in range(ROWS_PT // ZR):
            r0 = s * ROWS_PT + j * ZR
            pltpu.async_copy(acc.at[pl.ds(r0, ZR)],
                             out_hbm.at[c].at[pl.ds(r0, ZR)], gsem[j % NBUF])
        for j in range(ROWS_PT // ZR):
            r0 = s * ROWS_PT + j * ZR
            pltpu.make_async_copy(acc.at[pl.ds(r0, ZR)],
                                  out_hbm.at[c].at[pl.ds(r0, ZR)],
                                  gsem[j % NBUF]).wait()

    return k(p, src3, dst3, zeros_z)


_BR = 2000


def _tc_project0(x, W1):
    def body(x_ref, w_ref, o_ref):
        o_ref[...] = _dot(x_ref[...], w_ref[...])

    return pl.pallas_call(
        body,
        grid=(N // _BR,),
        in_specs=[
            pl.BlockSpec((_BR, DIN), lambda i: (i, 0)),
            pl.BlockSpec((DIN, H), lambda i: (0, 0)),
        ],
        out_specs=pl.BlockSpec((_BR, H), lambda i: (i, 0)),
        out_shape=jax.ShapeDtypeStruct((N, H), jnp.float32),
    )(x, W1)


def _tc_block(p, acc2, b1, W2, b2, g, be, W1n):
    def body(p_ref, a_ref, b1_ref, w2_ref, b2_ref, g_ref, be_ref, w1n_ref,
             o_ref):
        t = p_ref[...] + a_ref[0] + a_ref[1] + b1_ref[...]
        t = jnp.maximum(t, 0.0)
        u = _dot(t, w2_ref[...]) + b2_ref[...]
        mu = jnp.mean(u, axis=-1, keepdims=True)
        var = jnp.mean((u - mu) ** 2, axis=-1, keepdims=True)
        v = (u - mu) / jnp.sqrt(var + 1e-5) * g_ref[...] + be_ref[...]
        v = jnp.maximum(v, 0.0)
        o_ref[...] = _dot(v, w1n_ref[...])

    return pl.pallas_call(
        body,
        grid=(N // _BR,),
        in_specs=[
            pl.BlockSpec((_BR, H), lambda i: (i, 0)),
            pl.BlockSpec((NC, _BR, H), lambda i: (0, i, 0)),
            pl.BlockSpec((1, H), lambda i: (0, 0)),
            pl.BlockSpec((H, H), lambda i: (0, 0)),
            pl.BlockSpec((1, H), lambda i: (0, 0)),
            pl.BlockSpec((1, H), lambda i: (0, 0)),
            pl.BlockSpec((1, H), lambda i: (0, 0)),
            pl.BlockSpec((H, H), lambda i: (0, 0)),
        ],
        out_specs=pl.BlockSpec((_BR, H), lambda i: (i, 0)),
        out_shape=jax.ShapeDtypeStruct((N, H), jnp.float32),
    )(p, acc2, b1.reshape(1, H), W2, b2.reshape(1, H), g.reshape(1, H),
      be.reshape(1, H), W1n)


def _tc_final(p, acc2, b1, W2, b2):
    def body(p_ref, a_ref, b1_ref, w2_ref, b2_ref, o_ref):
        t = p_ref[...] + a_ref[0] + a_ref[1] + b1_ref[...]
        t = jnp.maximum(t, 0.0)
        o_ref[...] = _dot(t, w2_ref[...]) + b2_ref[...]

    return pl.pallas_call(
        body,
        grid=(N // _BR,),
        in_specs=[
            pl.BlockSpec((_BR, H), lambda i: (i, 0)),
            pl.BlockSpec((NC, _BR, H), lambda i: (0, i, 0)),
            pl.BlockSpec((1, H), lambda i: (0, 0)),
            pl.BlockSpec((H, DOUT), lambda i: (0, 0)),
            pl.BlockSpec((1, DOUT), lambda i: (0, 0)),
        ],
        out_specs=pl.BlockSpec((_BR, DOUT), lambda i: (i, 0)),
        out_shape=jax.ShapeDtypeStruct((N, DOUT), jnp.float32),
    )(p, acc2, b1.reshape(1, H), W2, b2.reshape(1, DOUT))


def kernel(x, edge_index,
           W1_0, b1_0, W2_0, b2_0, g_0, be_0,
           W1_1, b1_1, W2_1, b2_1, g_1, be_1,
           W1_2, b1_2, W2_2, b2_2, g_2, be_2,
           W1_3, b1_3, W2_3, b2_3):
    src3 = edge_index[0].astype(jnp.int32).reshape(NW, NCHUNKS, CHUNK)
    dst3 = edge_index[1].astype(jnp.int32).reshape(NW, NCHUNKS, CHUNK)
    zeros_z = jnp.zeros((ZR, H), jnp.float32)

    p = _tc_project0(x, W1_0)
    acc2 = _sc_segment_sum(p, src3, dst3, zeros_z)
    p = _tc_block(p, acc2, b1_0, W2_0, b2_0, g_0, be_0, W1_1)
    acc2 = _sc_segment_sum(p, src3, dst3, zeros_z)
    p = _tc_block(p, acc2, b1_1, W2_1, b2_1, g_1, be_1, W1_2)
    acc2 = _sc_segment_sum(p, src3, dst3, zeros_z)
    p = _tc_block(p, acc2, b1_2, W2_2, b2_2, g_2, be_2, W1_3)
    acc2 = _sc_segment_sum(p, src3, dst3, zeros_z)
    return _tc_final(p, acc2, b1_3, W2_3, b2_3)
